# Initial kernel scaffold; baseline (speedup 1.0000x reference)
import jax
import jax.numpy as jnp
from jax.experimental import pallas as pl


def kernel(user_index, user_table, item_table, edge_index, edge_weight):
    raise NotImplementedError("write your pallas kernel here")


# speedup vs baseline: 5.9838x; 5.9838x over previous
import functools

import jax
import jax.numpy as jnp
from jax import lax
from jax.experimental import pallas as pl
from jax.experimental.pallas import tpu as pltpu
from jax.experimental.pallas import tpu_sc as plsc

_NU = 50000
_NI = 50000
_NN = _NU + _NI
_D = 32
_NC, _NS = 2, 16
_CHK = 512
_SUBW = 128
_NSUB = _CHK // _SUBW
_HALF = _NN // _NC
_ACC_PAD = 50048
_ZPT = _ACC_PAD // _NS
_WBT = _HALF // _NS


def _prop_body(cur, srcm, dstm, wm, out, acc, rows, srcv, dstv, dstl, wv,
               gsem, ssem):
    c = lax.axis_index("c")
    s = lax.axis_index("s")
    rows_per_tile = srcm.shape[0] // _NS
    n_chunks = rows_per_tile * _SUBW // _CHK
    zero16 = jnp.zeros((16,), jnp.float32)

    def _z(i, carry):
        rows[i, 0:16] = zero16
        rows[i, 16:32] = zero16
        return carry
    lax.fori_loop(0, _CHK, _z, 0)

    def _zc(t, carry):
        pltpu.sync_copy(rows, acc.at[pl.ds(s * _ZPT + t * _CHK, _CHK)])
        return carry
    lax.fori_loop(0, _ZPT // _CHK, _zc, 0)
    zrem = _ZPT % _CHK
    pltpu.sync_copy(rows.at[pl.ds(0, zrem)],
                    acc.at[pl.ds(s * _ZPT + _ZPT - zrem, zrem)])
    plsc.subcore_barrier()

    half_base = c * _HALF

    def _chunk(k, carry):
        rb = s * rows_per_tile + k * (_CHK // _SUBW)
        pltpu.sync_copy(srcm.at[pl.ds(rb, _NSUB)], srcv)
        pltpu.sync_copy(dstm.at[pl.ds(rb, _NSUB)], dstv)
        pltpu.sync_copy(wm.at[pl.ds(rb, _NSUB)], wv)
        descs = [pltpu.async_copy(cur.at[srcv.at[j]],
                                  rows.at[pl.ds(j * _SUBW, _SUBW)], gsem)
                 for j in range(_NSUB)]
        for d in descs:
            d.wait()

        gdn = lax.GatherDimensionNumbers(offset_dims=(), collapsed_slice_dims=(0,),
                                         start_index_map=(0,))

        def _per_r(r, carry2):
            for u in range(8):
                v = dstv[r, pl.ds(u * 16, 16)]
                vl = v - half_base
                ok = (vl >= 0) & (vl < _HALF)
                dstl[r, pl.ds(u * 16, 16)] = jnp.where(ok, vl, _HALF)
                w16 = wv[r, pl.ds(u * 16, 16)]
                for i in range(16):
                    e = r * _SUBW + u * 16 + i
                    wb = lax.gather(
                        w16, jnp.full((16, 1), i, jnp.int32), gdn,
                        slice_sizes=(1,),
                        mode=lax.GatherScatterMode.PROMISE_IN_BOUNDS)
                    rows[e, 0:16] = rows[e, 0:16] * wb
                    rows[e, 16:32] = rows[e, 16:32] * wb
            return carry2
        lax.fori_loop(0, _NSUB, _per_r, 0)

        sds = [pltpu.async_copy(rows.at[pl.ds(j * _SUBW, _SUBW)],
                                acc.at[dstl.at[j]], ssem, add=True)
               for j in range(_NSUB)]
        for d in sds:
            d.wait()
        return carry
    lax.fori_loop(0, n_chunks, _chunk, 0)
    plsc.subcore_barrier()

    wb_l = s * _ZPT
    g = c * _HALF + wb_l

    def _wc(t, carry):
        pltpu.sync_copy(acc.at[pl.ds(wb_l + t * _CHK, _CHK)], rows)
        pltpu.sync_copy(rows, out.at[pl.ds(g + t * _CHK, _CHK)])
        return carry
    lax.fori_loop(0, 3080 // _CHK, _wc, 0)
    rem = 3080 % _CHK
    pltpu.sync_copy(acc.at[pl.ds(wb_l + 3080 - rem, rem)],
                    rows.at[pl.ds(0, rem)])
    pltpu.sync_copy(rows.at[pl.ds(0, rem)], out.at[pl.ds(g + 3080 - rem, rem)])

    @pl.when(s < _NS - 1)
    def _tail():
        pltpu.sync_copy(acc.at[pl.ds(wb_l + 3080, 48)],
                        rows.at[pl.ds(0, 48)])
        pltpu.sync_copy(rows.at[pl.ds(0, 48)], out.at[pl.ds(g + 3080, 48)])


def _users_body(t0, t1, t2, idx, o0, o1, o2, idxv, rowsv, sem):
    c = lax.axis_index("c")
    s = lax.axis_index("s")
    wid = s * _NC + c
    per = idx.shape[0] // (_NC * _NS)
    base = wid * per
    pltpu.sync_copy(idx.at[pl.ds(base, per)], idxv)
    for t, o in ((t0, o0), (t1, o1), (t2, o2)):
        pltpu.async_copy(t.at[idxv], rowsv, sem).wait()
        pltpu.sync_copy(rowsv, o.at[pl.ds(base, per)])


def _scores_body(u0, u1, u2, i0, i1, i2, out):
    dn = (((1,), (1,)), ((), ()))
    a0 = lax.dot_general(u0[...], i0[...], dn,
                         preferred_element_type=jnp.float32)
    a1 = lax.dot_general(u1[...], i1[...], dn,
                         preferred_element_type=jnp.float32)
    a2 = lax.dot_general(u2[...], i2[...], dn,
                         preferred_element_type=jnp.float32)
    acc = a0 * (1.0 / 9.0) + a1 * (4.0 / 9.0) + a2
    out[...] = 1.0 / (1.0 + jnp.exp(-acc))


def kernel(user_index, user_table, item_table, edge_index, edge_weight):
    num_users, d = user_table.shape
    num_items = item_table.shape[0]
    n = num_users + num_items
    batch = user_index.shape[0]
    e = edge_index.shape[1]

    e_pad = -(-e // (_NS * _CHK)) * (_NS * _CHK)
    src = jnp.pad(edge_index[0].astype(jnp.int32), (0, e_pad - e))
    dst = jnp.pad(edge_index[1].astype(jnp.int32), (0, e_pad - e))
    w = jnp.pad(edge_weight, (0, e_pad - e))
    srcm = src.reshape(e_pad // _SUBW, _SUBW)
    dstm = dst.reshape(e_pad // _SUBW, _SUBW)
    wm = w.reshape(e_pad // _SUBW, _SUBW)

    con0 = jnp.concatenate([user_table, item_table], axis=0)

    mesh = plsc.VectorSubcoreMesh(core_axis_name="c", subcore_axis_name="s",
                                  num_cores=_NC, num_subcores=_NS)
    sc_params = pltpu.CompilerParams(use_tc_tiling_on_sc=False)
    prop = pl.kernel(
        _prop_body,
        out_type=jax.ShapeDtypeStruct((n, d), jnp.float32),
        mesh=mesh,
        compiler_params=sc_params,
        scratch_types=[
            pltpu.VMEM_SHARED((_ACC_PAD, _D), jnp.float32),
            pltpu.VMEM((_CHK, _D), jnp.float32),
            pltpu.VMEM((_NSUB, _SUBW), jnp.int32),
            pltpu.VMEM((_NSUB, _SUBW), jnp.int32),
            pltpu.VMEM((_NSUB, _SUBW), jnp.int32),
            pltpu.VMEM((_NSUB, _SUBW), jnp.float32),
            pltpu.SemaphoreType.DMA,
            pltpu.SemaphoreType.DMA,
        ],
    )
    h1 = prop(con0, srcm, dstm, wm)
    h2 = prop(h1, srcm, dstm, wm)

    per = batch // (_NC * _NS)
    ug = pl.kernel(
        _users_body,
        out_type=[jax.ShapeDtypeStruct((batch, d), jnp.float32)] * 3,
        mesh=mesh,
        compiler_params=sc_params,
        scratch_types=[
            pltpu.VMEM((per,), jnp.int32),
            pltpu.VMEM((per, _D), jnp.float32),
            pltpu.SemaphoreType.DMA,
        ],
    )
    u0, u1, u2 = ug(con0, h1, h2, user_index.astype(jnp.int32))

    it0 = con0[num_users:]
    it1 = h1[num_users:]
    it2 = h2[num_users:]
    bi = 512
    gi = -(-num_items // bi)
    scores = pl.pallas_call(
        _scores_body,
        grid=(gi,),
        in_specs=[pl.BlockSpec((batch, d), lambda j: (0, 0))] * 3
        + [pl.BlockSpec((bi, d), lambda j: (j, 0))] * 3,
        out_specs=pl.BlockSpec((batch, bi), lambda j: (0, j)),
        out_shape=jax.ShapeDtypeStruct((batch, num_items), jnp.float32),
    )(u0, u1, u2, it0, it1, it2)
    return scores


# speedup vs baseline: 7.0923x; 1.1853x over previous
import functools

import jax
import jax.numpy as jnp
from jax import lax
from jax.experimental import pallas as pl
from jax.experimental.pallas import tpu as pltpu
from jax.experimental.pallas import tpu_sc as plsc

_NU = 50000
_NI = 50000
_NN = _NU + _NI
_D = 32
_NC, _NS = 2, 16
_HD = _D // _NC
_CHK = 512
_SUBW = 128
_NSUB = _CHK // _SUBW
_ACC_PAD = 100096
_ZPT = _ACC_PAD // _NS


def _prop_body(cur, srcm2, dstm, wm, out, acc, rows, srcv, dstv, wv,
               gsem, ssem):
    c = lax.axis_index("c")
    s = lax.axis_index("s")
    rows_per_tile = srcm2.shape[1] // _NS
    n_chunks = rows_per_tile * _SUBW // _CHK
    zero16 = jnp.zeros((16,), jnp.float32)

    def _z(i, carry):
        rows[i, 0:16] = zero16
        return carry
    lax.fori_loop(0, _CHK, _z, 0)

    def _zc(t, carry):
        pltpu.sync_copy(rows, acc.at[pl.ds(s * _ZPT + t * _CHK, _CHK)])
        return carry
    lax.fori_loop(0, _ZPT // _CHK, _zc, 0)
    zrem = _ZPT % _CHK
    pltpu.sync_copy(rows.at[pl.ds(0, zrem)],
                    acc.at[pl.ds(s * _ZPT + _ZPT - zrem, zrem)])
    plsc.subcore_barrier()

    gdn = lax.GatherDimensionNumbers(offset_dims=(), collapsed_slice_dims=(0,),
                                     start_index_map=(0,))

    def _chunk(k, carry):
        rb = s * rows_per_tile + k * _NSUB
        pltpu.sync_copy(srcm2.at[c, pl.ds(rb, _NSUB)], srcv)
        pltpu.sync_copy(dstm.at[pl.ds(rb, _NSUB)], dstv)
        pltpu.sync_copy(wm.at[pl.ds(rb, _NSUB)], wv)
        descs = [pltpu.async_copy(cur.at[srcv.at[j]],
                                  rows.at[pl.ds(j * _SUBW, _SUBW)], gsem)
                 for j in range(_NSUB)]
        for dsc in descs:
            dsc.wait()

        def _per_r(r, carry2):
            for u in range(8):
                w16 = wv[r, pl.ds(u * 16, 16)]
                for i in range(16):
                    e = r * _SUBW + u * 16 + i
                    wb = lax.gather(
                        w16, jnp.full((16, 1), i, jnp.int32), gdn,
                        slice_sizes=(1,),
                        mode=lax.GatherScatterMode.PROMISE_IN_BOUNDS)
                    rows[e, 0:16] = rows[e, 0:16] * wb
            return carry2
        lax.fori_loop(0, _NSUB, _per_r, 0)

        sds = [pltpu.async_copy(rows.at[pl.ds(j * _SUBW, _SUBW)],
                                acc.at[dstv.at[j]], ssem, add=True)
               for j in range(_NSUB)]
        for dsc in sds:
            dsc.wait()
        return carry
    lax.fori_loop(0, n_chunks, _chunk, 0)
    plsc.subcore_barrier()

    wb_l = s * _ZPT
    ob = c * _NN + wb_l
    nfull = 6160 // _CHK
    wrem = 6160 % _CHK

    def _wc(t, carry):
        pltpu.sync_copy(acc.at[pl.ds(wb_l + t * _CHK, _CHK)], rows)
        pltpu.sync_copy(rows, out.at[pl.ds(ob + t * _CHK, _CHK)])
        return carry
    lax.fori_loop(0, nfull, _wc, 0)
    pltpu.sync_copy(acc.at[pl.ds(wb_l + 6160 - wrem, wrem)],
                    rows.at[pl.ds(0, wrem)])
    pltpu.sync_copy(rows.at[pl.ds(0, wrem)],
                    out.at[pl.ds(ob + 6160 - wrem, wrem)])

    @pl.when(s < _NS - 1)
    def _tail():
        pltpu.sync_copy(acc.at[pl.ds(wb_l + 6160, 96)], rows.at[pl.ds(0, 96)])
        pltpu.sync_copy(rows.at[pl.ds(0, 96)], out.at[pl.ds(ob + 6160, 96)])


def _users_body(t0, t1, t2, idx, o0, o1, o2, idxv, idxv2, rowsv, sem):
    c = lax.axis_index("c")
    s = lax.axis_index("s")
    wid = s * _NC + c
    per = idx.shape[0] // (_NC * _NS)
    base = wid * per
    pltpu.sync_copy(idx.at[pl.ds(base, per)], idxv)
    for q in range(per // 16):
        idxv2[pl.ds(q * 16, 16)] = idxv[pl.ds(q * 16, 16)] + _NN
    for t, o in ((t0, o0), (t1, o1), (t2, o2)):
        pltpu.async_copy(t.at[idxv], rowsv, sem).wait()
        pltpu.sync_copy(rowsv, o.at[pl.ds(base, per), pl.ds(0, _HD)])
        pltpu.async_copy(t.at[idxv2], rowsv, sem).wait()
        pltpu.sync_copy(rowsv, o.at[pl.ds(base, per), pl.ds(_HD, _HD)])


def _scores_body(u, it, out):
    dn = (((1,), (1,)), ((), ()))
    acc = lax.dot_general(u[...], it[...], dn,
                          preferred_element_type=jnp.float32)
    out[...] = 1.0 / (1.0 + jnp.exp(-acc))


def kernel(user_index, user_table, item_table, edge_index, edge_weight):
    num_users, d = user_table.shape
    num_items = item_table.shape[0]
    n = num_users + num_items
    batch = user_index.shape[0]
    e = edge_index.shape[1]

    e_pad = -(-e // (_NS * _CHK)) * (_NS * _CHK)
    src = jnp.pad(edge_index[0].astype(jnp.int32), (0, e_pad - e))
    dst = jnp.pad(edge_index[1].astype(jnp.int32), (0, e_pad - e))
    w = jnp.pad(edge_weight, (0, e_pad - e))
    srcm2 = jnp.stack([src, src + n]).reshape(2, e_pad // _SUBW, _SUBW)
    dstm = dst.reshape(e_pad // _SUBW, _SUBW)
    wm = w.reshape(e_pad // _SUBW, _SUBW)

    con0 = jnp.concatenate(
        [user_table[:, :_HD], item_table[:, :_HD],
         user_table[:, _HD:], item_table[:, _HD:]], axis=0)

    mesh = plsc.VectorSubcoreMesh(core_axis_name="c", subcore_axis_name="s",
                                  num_cores=_NC, num_subcores=_NS)
    sc_params = pltpu.CompilerParams(use_tc_tiling_on_sc=False)
    prop = pl.kernel(
        _prop_body,
        out_type=jax.ShapeDtypeStruct((2 * n, _HD), jnp.float32),
        mesh=mesh,
        compiler_params=sc_params,
        scratch_types=[
            pltpu.VMEM_SHARED((_ACC_PAD, _HD), jnp.float32),
            pltpu.VMEM((_CHK, _HD), jnp.float32),
            pltpu.VMEM((_NSUB, _SUBW), jnp.int32),
            pltpu.VMEM((_NSUB, _SUBW), jnp.int32),
            pltpu.VMEM((_NSUB, _SUBW), jnp.float32),
            pltpu.SemaphoreType.DMA,
            pltpu.SemaphoreType.DMA,
        ],
    )
    h1 = prop(con0, srcm2, dstm, wm)
    h2 = prop(h1, srcm2, dstm, wm)

    per = batch // (_NC * _NS)
    ug = pl.kernel(
        _users_body,
        out_type=[jax.ShapeDtypeStruct((batch, _D), jnp.float32)] * 3,
        mesh=mesh,
        compiler_params=sc_params,
        scratch_types=[
            pltpu.VMEM((per,), jnp.int32),
            pltpu.VMEM((per,), jnp.int32),
            pltpu.VMEM((per, _HD), jnp.float32),
            pltpu.SemaphoreType.DMA,
        ],
    )
    u0, u1, u2 = ug(con0, h1, h2, user_index.astype(jnp.int32))

    lam = [1.0 / 9.0, 4.0 / 9.0, 1.0]
    u = jnp.concatenate([u0 * lam[0], u1 * lam[1], u2 * lam[2]], axis=1)
    it = jnp.concatenate(
        [con0[num_users:n], con0[n + num_users:],
         h1[num_users:n], h1[n + num_users:],
         h2[num_users:n], h2[n + num_users:]], axis=1)
    bi = 1024
    gi = -(-num_items // bi)
    scores = pl.pallas_call(
        _scores_body,
        grid=(gi,),
        in_specs=[pl.BlockSpec((batch, 3 * _D), lambda j: (0, 0)),
                  pl.BlockSpec((bi, 3 * _D), lambda j: (j, 0))],
        out_specs=pl.BlockSpec((batch, bi), lambda j: (0, j)),
        out_shape=jax.ShapeDtypeStruct((batch, num_items), jnp.float32),
    )(u, it)
    return scores


# speedup vs baseline: 11.1315x; 1.5695x over previous
import functools

import jax
import jax.numpy as jnp
from jax import lax
from jax.experimental import pallas as pl
from jax.experimental.pallas import tpu as pltpu
from jax.experimental.pallas import tpu_sc as plsc

_NU = 50000
_NI = 50000
_NN = _NU + _NI
_D = 32
_NC, _NS = 2, 16
_HD = _D // _NC
_CHK = 256
_SUBW = 128
_NSUB = _CHK // _SUBW
_EVR = 2 * _NSUB
_NQ = 4
_ACC_PAD = 100096
_ZPT = _ACC_PAD // _NS


def _prop_body(cur, evm, wm, out, acc,
               rows0, rows1, rows2, rows3,
               evb0, evb1, evb2, evb3,
               wvb0, wvb1, wvb2, wvb3,
               isem, gsem, ssem):
    c = lax.axis_index("c")
    s = lax.axis_index("s")
    npt = evm.shape[1] // _EVR // _NS
    zero16 = jnp.zeros((16,), jnp.float32)
    rowsb = (rows0, rows1, rows2, rows3)
    evbb = (evb0, evb1, evb2, evb3)
    wvbb = (wvb0, wvb1, wvb2, wvb3)

    def _z(i, carry):
        rows0[i, 0:16] = zero16
        return carry
    lax.fori_loop(0, _CHK, _z, 0)

    def _zc(t, carry):
        pltpu.sync_copy(rows0, acc.at[pl.ds(s * _ZPT + t * _CHK, _CHK)])
        return carry
    lax.fori_loop(0, _ZPT // _CHK, _zc, 0)
    zrem = _ZPT % _CHK
    pltpu.sync_copy(rows0.at[pl.ds(0, zrem)],
                    acc.at[pl.ds(s * _ZPT + _ZPT - zrem, zrem)])
    plsc.subcore_barrier()

    gdn = lax.GatherDimensionNumbers(offset_dims=(), collapsed_slice_dims=(0,),
                                     start_index_map=(0,))

    def _step(g, carry):
        base = g * _NQ
        evds = []
        for q in range(_NQ):
            kq = base + q
            rb = (s * npt + kq) * _EVR
            wb = (s * npt + kq) * _NSUB
            evds.append(
                (pltpu.async_copy(evm.at[c, pl.ds(rb, _EVR)], evbb[q], isem),
                 pltpu.async_copy(wm.at[pl.ds(wb, _NSUB)], wvbb[q], isem)))
        gds = []
        for q in range(_NQ):
            for dsc in evds[q]:
                dsc.wait()
            gds.append([
                pltpu.async_copy(cur.at[evbb[q].at[j]],
                                 rowsb[q].at[pl.ds(j * _SUBW, _SUBW)], gsem)
                for j in range(_NSUB)])
        sds = []
        for q in range(_NQ):
            for dsc in gds[q]:
                dsc.wait()

            def _per4(r4, carry2, q=q):
                for u in range(8):
                    w16 = wvbb[q][r4, pl.ds(u * 16, 16)]
                    for i in range(16):
                        e = r4 * _SUBW + u * 16 + i
                        wb2 = lax.gather(
                            w16, jnp.full((16, 1), i, jnp.int32), gdn,
                            slice_sizes=(1,),
                            mode=lax.GatherScatterMode.PROMISE_IN_BOUNDS)
                        rowsb[q][e, 0:16] = rowsb[q][e, 0:16] * wb2
                return carry2
            lax.fori_loop(0, _NSUB, _per4, 0)

            sds.append([
                pltpu.async_copy(rowsb[q].at[pl.ds(j * _SUBW, _SUBW)],
                                 acc.at[evbb[q].at[_NSUB + j]], ssem,
                                 add=True)
                for j in range(_NSUB)])
        for q in range(_NQ):
            for dsc in sds[q]:
                dsc.wait()
        return carry
    lax.fori_loop(0, npt // _NQ, _step, 0)
    plsc.subcore_barrier()

    wb_l = s * _ZPT
    ob = c * _NN + wb_l
    nfull = 6160 // _CHK
    wrem = 6160 % _CHK

    def _wc(t, carry):
        pltpu.sync_copy(acc.at[pl.ds(wb_l + t * _CHK, _CHK)], rows0)
        pltpu.sync_copy(rows0, out.at[pl.ds(ob + t * _CHK, _CHK)])
        return carry
    lax.fori_loop(0, nfull, _wc, 0)
    pltpu.sync_copy(acc.at[pl.ds(wb_l + 6160 - wrem, wrem)],
                    rows0.at[pl.ds(0, wrem)])
    pltpu.sync_copy(rows0.at[pl.ds(0, wrem)],
                    out.at[pl.ds(ob + 6160 - wrem, wrem)])

    @pl.when(s < _NS - 1)
    def _tail():
        pltpu.sync_copy(acc.at[pl.ds(wb_l + 6160, 96)],
                        rows0.at[pl.ds(0, 96)])
        pltpu.sync_copy(rows0.at[pl.ds(0, 96)], out.at[pl.ds(ob + 6160, 96)])


def _users_body(t0, t1, t2, idx, o0, o1, o2, idxv, idxv2, rowsv, sem):
    c = lax.axis_index("c")
    s = lax.axis_index("s")
    wid = s * _NC + c
    per = idx.shape[0] // (_NC * _NS)
    base = wid * per
    pltpu.sync_copy(idx.at[pl.ds(base, per)], idxv)
    for q in range(per // 16):
        idxv2[pl.ds(q * 16, 16)] = idxv[pl.ds(q * 16, 16)] + _NN
    for t, o in ((t0, o0), (t1, o1), (t2, o2)):
        pltpu.async_copy(t.at[idxv], rowsv, sem).wait()
        pltpu.sync_copy(rowsv, o.at[pl.ds(base, per), pl.ds(0, _HD)])
        pltpu.async_copy(t.at[idxv2], rowsv, sem).wait()
        pltpu.sync_copy(rowsv, o.at[pl.ds(base, per), pl.ds(_HD, _HD)])


def _scores_body(u, it, out):
    dn = (((1,), (1,)), ((), ()))
    acc = lax.dot_general(u[...], it[...], dn,
                          preferred_element_type=jnp.float32)
    out[...] = 1.0 / (1.0 + jnp.exp(-acc))


def kernel(user_index, user_table, item_table, edge_index, edge_weight):
    num_users, d = user_table.shape
    num_items = item_table.shape[0]
    n = num_users + num_items
    batch = user_index.shape[0]
    e = edge_index.shape[1]

    e_pad = -(-e // (_NS * _NQ * _CHK)) * (_NS * _NQ * _CHK)
    src = jnp.pad(edge_index[0].astype(jnp.int32), (0, e_pad - e))
    dst = jnp.pad(edge_index[1].astype(jnp.int32), (0, e_pad - e))
    w = jnp.pad(edge_weight, (0, e_pad - e))
    dstr = dst.reshape(-1, _NSUB, _SUBW)
    evm = jnp.stack([
        jnp.concatenate([src.reshape(-1, _NSUB, _SUBW), dstr], axis=1),
        jnp.concatenate([(src + n).reshape(-1, _NSUB, _SUBW), dstr], axis=1),
    ]).reshape(2, -1, _SUBW)
    wm = w.reshape(-1, _SUBW)

    con0 = jnp.concatenate(
        [user_table[:, :_HD], item_table[:, :_HD],
         user_table[:, _HD:], item_table[:, _HD:]], axis=0)

    mesh = plsc.VectorSubcoreMesh(core_axis_name="c", subcore_axis_name="s",
                                  num_cores=_NC, num_subcores=_NS)
    sc_params = pltpu.CompilerParams(use_tc_tiling_on_sc=False)
    prop = pl.kernel(
        _prop_body,
        out_type=jax.ShapeDtypeStruct((2 * n, _HD), jnp.float32),
        mesh=mesh,
        compiler_params=sc_params,
        scratch_types=(
            [pltpu.VMEM_SHARED((_ACC_PAD, _HD), jnp.float32)]
            + [pltpu.VMEM((_CHK, _HD), jnp.float32)] * _NQ
            + [pltpu.VMEM((_EVR, _SUBW), jnp.int32)] * _NQ
            + [pltpu.VMEM((_NSUB, _SUBW), jnp.float32)] * _NQ
            + [pltpu.SemaphoreType.DMA] * 3
        ),
    )
    h1 = prop(con0, evm, wm)
    h2 = prop(h1, evm, wm)

    per = batch // (_NC * _NS)
    ug = pl.kernel(
        _users_body,
        out_type=[jax.ShapeDtypeStruct((batch, _D), jnp.float32)] * 3,
        mesh=mesh,
        compiler_params=sc_params,
        scratch_types=[
            pltpu.VMEM((per,), jnp.int32),
            pltpu.VMEM((per,), jnp.int32),
            pltpu.VMEM((per, _HD), jnp.float32),
            pltpu.SemaphoreType.DMA,
        ],
    )
    u0, u1, u2 = ug(con0, h1, h2, user_index.astype(jnp.int32))

    lam = [1.0 / 9.0, 4.0 / 9.0, 1.0]
    u = jnp.concatenate([u0 * lam[0], u1 * lam[1], u2 * lam[2]],
                        axis=1).astype(jnp.bfloat16)
    it = jnp.concatenate(
        [con0[num_users:n], con0[n + num_users:],
         h1[num_users:n], h1[n + num_users:],
         h2[num_users:n], h2[n + num_users:]], axis=1).astype(jnp.bfloat16)
    bi = 2048
    gi = -(-num_items // bi)
    scores = pl.pallas_call(
        _scores_body,
        grid=(gi,),
        in_specs=[pl.BlockSpec((batch, 3 * _D), lambda j: (0, 0)),
                  pl.BlockSpec((bi, 3 * _D), lambda j: (j, 0))],
        out_specs=pl.BlockSpec((batch, bi), lambda j: (0, j)),
        out_shape=jax.ShapeDtypeStruct((batch, num_items), jnp.float32),
    )(u, it)
    return scores
